# baseline (speedup 1.0000x reference)
import functools

import jax
import jax.numpy as jnp
from jax import lax
from jax.experimental import pallas as pl
from jax.experimental.pallas import tpu as pltpu
from jax.experimental.pallas import tpu_sc as plsc

F32 = jnp.float32
H = 128
N_NODES = 10000
N_PAD = 10240
E = 320000
E_CHUNK = 128
N_TILES = 32
CT = 80
IG = 16
E_PAD = N_TILES * CT * E_CHUNK
CNT_LEN = 10240
EL = 100000
CTL = 25
EL_PAD = N_TILES * CTL * E_CHUNK

_mesh = plsc.VectorSubcoreMesh(core_axis_name="c", subcore_axis_name="s")


CH = 64
SEG_CT = 320
SEG_IG = 8


def _seg_body(gidx, sidx, table, zacc, padidx, out_s,
              gv, sv, r0, r1, r2, r3, padv, acc,
              g0, g1, g2, g3, s0, s1, s2, s3):
    cid = lax.axis_index("c")
    sid = lax.axis_index("s")
    base = cid * (16 * SEG_CT) + sid * SEG_CT
    stripe = N_PAD // 16
    pltpu.sync_copy(zacc.at[pl.ds(sid * stripe, stripe)],
                    acc.at[pl.ds(sid * stripe, stripe)])
    pltpu.sync_copy(padidx, padv)
    plsc.subcore_barrier()
    bufs = ((r0, g0, s0), (r1, g1, s1), (r2, g2, s2), (r3, g3, s3))
    for (rb, _, sb) in bufs:
        pltpu.async_copy(rb, acc.at[padv], sb, add=True)

    def group(g, carry):
        pltpu.sync_copy(gidx.at[pl.ds(base + g * SEG_IG, SEG_IG)], gv)
        pltpu.sync_copy(sidx.at[pl.ds(base + g * SEG_IG, SEG_IG)], sv)
        for k in (0, 1):
            rb, gb, sb = bufs[k]
            pltpu.make_async_copy(rb, acc.at[padv], sb).wait()
            pltpu.async_copy(table.at[gv.at[k]], rb, gb)
        for pp in range(4):
            cur = bufs[:2] if pp % 2 == 0 else bufs[2:]
            nxt = bufs[2:] if pp % 2 == 0 else bufs[:2]
            if pp < 3:
                for k in (0, 1):
                    rb, gb, sb = nxt[k]
                    pltpu.make_async_copy(rb, acc.at[padv], sb).wait()
                    pltpu.async_copy(table.at[gv.at[2 * pp + 2 + k]], rb, gb)
            for k in (0, 1):
                rb, gb, sb = cur[k]
                pltpu.make_async_copy(table.at[gv.at[2 * pp + k]],
                                      rb, gb).wait()
                pltpu.async_copy(rb, acc.at[sv.at[2 * pp + k]], sb, add=True)
        return carry
    lax.fori_loop(0, SEG_CT // SEG_IG, group, 0)
    for (rb, _, sb) in bufs:
        pltpu.make_async_copy(rb, acc.at[padv], sb).wait()
    plsc.subcore_barrier()
    pltpu.sync_copy(acc.at[pl.ds(sid * stripe, stripe)],
                    out_s.at[pl.ds(cid * N_PAD + sid * stripe, stripe)])


_seg_call = functools.partial(
    pl.kernel, mesh=_mesh,
    compiler_params=pltpu.CompilerParams(use_tc_tiling_on_sc=False),
    out_type=[jax.ShapeDtypeStruct((2 * N_PAD, H), F32)],
    scratch_types=[
        pltpu.VMEM((SEG_IG, CH), jnp.int32),
        pltpu.VMEM((SEG_IG, CH), jnp.int32),
        pltpu.VMEM((CH, H), F32),
        pltpu.VMEM((CH, H), F32),
        pltpu.VMEM((CH, H), F32),
        pltpu.VMEM((CH, H), F32),
        pltpu.VMEM((CH,), jnp.int32),
        pltpu.VMEM_SHARED((N_PAD, H), F32),
        pltpu.SemaphoreType.DMA,
        pltpu.SemaphoreType.DMA,
        pltpu.SemaphoreType.DMA,
        pltpu.SemaphoreType.DMA,
        pltpu.SemaphoreType.DMA,
        pltpu.SemaphoreType.DMA,
        pltpu.SemaphoreType.DMA,
        pltpu.SemaphoreType.DMA,
    ],
)(_seg_body)


def _cnt_body(sidx, zacc, ones_in, out_c, sv, onesv, acc, sem):
    cid = lax.axis_index("c")
    sid = lax.axis_index("s")
    base = cid * (16 * SEG_CT) + sid * SEG_CT
    stripe = N_PAD // 16
    pltpu.sync_copy(zacc.at[pl.ds(sid * stripe, stripe)],
                    acc.at[pl.ds(sid * stripe, stripe)])
    pltpu.sync_copy(ones_in, onesv)
    plsc.subcore_barrier()

    def group(g, carry):
        pltpu.sync_copy(sidx.at[pl.ds(base + g * SEG_IG, SEG_IG)], sv)

        def issue(j, carry2):
            pltpu.async_copy(onesv, acc.at[sv.at[j]], sem, add=True)
            return carry2
        lax.fori_loop(0, SEG_IG, issue, 0)

        def drain(j, carry2):
            pltpu.make_async_copy(onesv, acc.at[sv.at[j]], sem).wait()
            return carry2
        lax.fori_loop(0, SEG_IG, drain, 0)
        return carry
    lax.fori_loop(0, SEG_CT // SEG_IG, group, 0)
    plsc.subcore_barrier()
    pltpu.sync_copy(acc.at[pl.ds(sid * stripe, stripe)],
                    out_c.at[pl.ds(cid * N_PAD + sid * stripe, stripe)])


_cnt_call = functools.partial(
    pl.kernel, mesh=_mesh,
    out_type=[jax.ShapeDtypeStruct((2 * N_PAD, H), F32)],
    scratch_types=[
        pltpu.VMEM((SEG_IG, CH), jnp.int32),
        pltpu.VMEM((CH, H), F32),
        pltpu.VMEM_SHARED((N_PAD, H), F32),
        pltpu.SemaphoreType.DMA,
    ],
)(_cnt_body)


def _cls_body(aidx, bidx, ta, tb, out, av, bv, ar, br, res, sema, semb):
    cid = lax.axis_index("c")
    sid = lax.axis_index("s")
    wid = cid * 16 + sid
    pltpu.sync_copy(aidx.at[pl.ds(wid * 32, 32)], av)
    pltpu.sync_copy(bidx.at[pl.ds(wid * 32, 32)], bv)

    def chunk(j, carry):
        ca = pltpu.async_copy(ta.at[av.at[j]], ar, sema)
        cb = pltpu.async_copy(tb.at[bv.at[j]], br, semb)
        ca.wait()
        cb.wait()

        def row(r, c2):
            acc = ar[r, pl.ds(0, 16)] * br[r, pl.ds(0, 16)]
            for k in range(1, 8):
                acc = acc + ar[r, pl.ds(k * 16, 16)] * br[r, pl.ds(k * 16, 16)]
            res[r] = acc
            return c2
        lax.fori_loop(0, E_CHUNK, row, 0)
        pltpu.sync_copy(res, out.at[pl.ds(wid * CTL * E_CHUNK + j * E_CHUNK,
                                          E_CHUNK)])
        return carry
    lax.fori_loop(0, CTL, chunk, 0)


_cls_call = functools.partial(
    pl.kernel, mesh=_mesh,
    out_type=[jax.ShapeDtypeStruct((EL_PAD, 16), F32)],
    scratch_types=[
        pltpu.VMEM((32, E_CHUNK), jnp.int32),
        pltpu.VMEM((32, E_CHUNK), jnp.int32),
        pltpu.VMEM((E_CHUNK, H), F32),
        pltpu.VMEM((E_CHUNK, H), F32),
        pltpu.VMEM((E_CHUNK, 16), F32),
        pltpu.SemaphoreType.DMA,
        pltpu.SemaphoreType.DMA,
    ],
)(_cls_body)


def _cls_reduce(p):
    def body(p_ref, o_ref):
        o_ref[...] = jnp.sum(p_ref[...], axis=1)
    return pl.pallas_call(
        body,
        grid=(100,),
        in_specs=[pl.BlockSpec((1024, 16), lambda i: (i, 0))],
        out_specs=pl.BlockSpec((1024,), lambda i: (i,)),
        out_shape=jax.ShapeDtypeStruct((EL_PAD,), F32),
    )(p)


def _encode(x, wt, b2, emb):
    def body(x_ref, w_ref, b_ref, e_ref, o_ref):
        o_ref[...] = (jnp.dot(x_ref[...], w_ref[...],
                              preferred_element_type=F32)
                      + b_ref[...] + e_ref[...])
    return pl.pallas_call(
        body,
        grid=(10,),
        in_specs=[pl.BlockSpec((1000, H), lambda i: (i, 0)),
                  pl.BlockSpec((H, H), lambda i: (0, 0)),
                  pl.BlockSpec((1, H), lambda i: (0, 0)),
                  pl.BlockSpec((1000, H), lambda i: (i, 0))],
        out_specs=pl.BlockSpec((1000, H), lambda i: (i, 0)),
        out_shape=jax.ShapeDtypeStruct((N_NODES, H), F32),
    )(x, wt, b2, emb)


def _combine(sums, counts, h_self, wlt, bl2, wrt, relu):
    def body(s_ref, c_ref, h_ref, wl_ref, b_ref, wr_ref, o_ref):
        c = c_ref[:, 0]
        r = 1.0 / jnp.maximum(c, 1.0)
        agg = s_ref[...] * r[:, None]
        y = (jnp.dot(agg, wl_ref[...], preferred_element_type=F32)
             + b_ref[...]
             + jnp.dot(h_ref[...], wr_ref[...], preferred_element_type=F32))
        o_ref[...] = jnp.maximum(y, 0.0) if relu else y
    return pl.pallas_call(
        body,
        grid=(10,),
        in_specs=[pl.BlockSpec((1000, H), lambda i: (i, 0)),
                  pl.BlockSpec((1000, H), lambda i: (i, 0)),
                  pl.BlockSpec((1000, H), lambda i: (i, 0)),
                  pl.BlockSpec((H, H), lambda i: (0, 0)),
                  pl.BlockSpec((1, H), lambda i: (0, 0)),
                  pl.BlockSpec((H, H), lambda i: (0, 0))],
        out_specs=pl.BlockSpec((1000, H), lambda i: (i, 0)),
        out_shape=jax.ShapeDtypeStruct((N_NODES, H), F32),
    )(sums, counts, h_self, wlt, bl2, wrt)


def kernel(x_user, x_movie, user_node_id, movie_node_id, edge_index,
           edge_label_index, W_user_lin, b_user_lin, W_movie_lin, b_movie_lin,
           user_emb, movie_emb, Wl1r, bl1r, Wr1r, Wl1v, bl1v, Wr1v,
           Wl2r, bl2r, Wr2r, Wl2v, bl2v, Wr2v):
    src = edge_index[0]
    dst = edge_index[1]
    padg = jnp.zeros((E_PAD - E,), jnp.int32)
    pads = jnp.full((E_PAD - E,), N_NODES, jnp.int32)
    g1 = jnp.concatenate([src, padg])
    s1 = jnp.concatenate([dst, pads])
    g2 = jnp.concatenate([dst, padg]) + N_NODES
    s2 = jnp.concatenate([src, pads])
    g_all = jnp.concatenate([g1, g2]).reshape(2 * 16 * SEG_CT, CH)
    s_all = jnp.concatenate([s1, s2]).reshape(2 * 16 * SEG_CT, CH)
    padl = jnp.zeros((EL_PAD - EL,), jnp.int32)

    def _lidx(v):
        m = jnp.concatenate([v, padl]).reshape(N_TILES, CTL, E_CHUNK)
        return jnp.pad(m, ((0, 0), (0, 32 - CTL), (0, 0))).reshape(
            N_TILES * 32, E_CHUNK)
    la = _lidx(edge_label_index[0])
    lb = _lidx(edge_label_index[1])
    zacc = jnp.zeros((N_PAD, H), F32)
    ones_rows = jnp.ones((CH, H), F32)
    padidx = jnp.full((CH,), N_NODES, jnp.int32)

    hu = _encode(x_user, W_user_lin.T, b_user_lin.reshape(1, H), user_emb)
    hm = _encode(x_movie, W_movie_lin.T, b_movie_lin.reshape(1, H), movie_emb)

    (s12,) = _seg_call(g_all, s_all, jnp.concatenate([hu, hm]), zacc, padidx)
    (c12,) = _cnt_call(s_all, zacc, ones_rows)
    sm = s12[:N_NODES]
    su = s12[N_PAD:N_PAD + N_NODES]
    cm = c12[:N_NODES]
    cu = c12[N_PAD:N_PAD + N_NODES]

    hm1 = _combine(sm, cm, hm, Wl1r.T, bl1r.reshape(1, H), Wr1r.T, True)
    hu1 = _combine(su, cu, hu, Wl1v.T, bl1v.reshape(1, H), Wr1v.T, True)

    (s34,) = _seg_call(g_all, s_all, jnp.concatenate([hu1, hm1]), zacc,
                       padidx)
    sm2 = s34[:N_NODES]
    su2 = s34[N_PAD:N_PAD + N_NODES]

    hm2 = _combine(sm2, cm, hm1, Wl2r.T, bl2r.reshape(1, H), Wr2r.T, False)
    hu2 = _combine(su2, cu, hu1, Wl2v.T, bl2v.reshape(1, H), Wr2v.T, False)

    (partials,) = _cls_call(la, lb, hu2, hm2)
    out = _cls_reduce(partials)
    return out[:EL]


# speedup vs baseline: 1.0520x; 1.0520x over previous
import functools

import jax
import jax.numpy as jnp
from jax import lax
from jax.experimental import pallas as pl
from jax.experimental.pallas import tpu as pltpu
from jax.experimental.pallas import tpu_sc as plsc

F32 = jnp.float32
H = 128
N_NODES = 10000
N_PAD = 10240
E = 320000
CH = 64
SEG_CT = 320
SEG_IG = 8
E_PAD = 16 * SEG_CT * CH
N_TILES = 32
EL = 100000
E_CHUNK = 128
CTL = 25
EL_PAD = N_TILES * CTL * E_CHUNK

_mesh = plsc.VectorSubcoreMesh(core_axis_name="c", subcore_axis_name="s")


def _seg_body(gidx, sidx, table, zacc, padidx, out_s,
              gv, sv, r0, r1, r2, r3, padv, acc,
              g0, g1, g2, g3, s0, s1, s2, s3):
    cid = lax.axis_index("c")
    sid = lax.axis_index("s")
    base = cid * (16 * SEG_CT) + sid * SEG_CT
    stripe = N_PAD // 16
    pltpu.sync_copy(zacc.at[pl.ds(sid * stripe, stripe)],
                    acc.at[pl.ds(sid * stripe, stripe)])
    pltpu.sync_copy(padidx, padv)
    plsc.subcore_barrier()
    bufs = ((r0, g0, s0), (r1, g1, s1), (r2, g2, s2), (r3, g3, s3))
    for (rb, _, sb) in bufs:
        pltpu.async_copy(rb, acc.at[padv], sb, add=True)

    def group(g, carry):
        pltpu.sync_copy(gidx.at[pl.ds(base + g * SEG_IG, SEG_IG)], gv)
        pltpu.sync_copy(sidx.at[pl.ds(base + g * SEG_IG, SEG_IG)], sv)
        for k in (0, 1):
            rb, gb, sb = bufs[k]
            pltpu.make_async_copy(rb, acc.at[padv], sb).wait()
            pltpu.async_copy(table.at[gv.at[k]], rb, gb)
        for pp in range(4):
            cur = bufs[:2] if pp % 2 == 0 else bufs[2:]
            nxt = bufs[2:] if pp % 2 == 0 else bufs[:2]
            if pp < 3:
                for k in (0, 1):
                    rb, gb, sb = nxt[k]
                    pltpu.make_async_copy(rb, acc.at[padv], sb).wait()
                    pltpu.async_copy(table.at[gv.at[2 * pp + 2 + k]], rb, gb)
            for k in (0, 1):
                rb, gb, sb = cur[k]
                pltpu.make_async_copy(table.at[gv.at[2 * pp + k]],
                                      rb, gb).wait()
                pltpu.async_copy(rb, acc.at[sv.at[2 * pp + k]], sb, add=True)
        return carry
    lax.fori_loop(0, SEG_CT // SEG_IG, group, 0)
    for (rb, _, sb) in bufs:
        pltpu.make_async_copy(rb, acc.at[padv], sb).wait()
    plsc.subcore_barrier()
    obase = (1 - cid) * N_NODES

    @pl.when(sid < 15)
    def _():
        pltpu.sync_copy(acc.at[pl.ds(sid * stripe, stripe)],
                        out_s.at[pl.ds(obase + sid * stripe, stripe)])

    @pl.when(sid == 15)
    def _():
        pltpu.sync_copy(acc.at[pl.ds(15 * stripe, N_NODES - 15 * stripe)],
                        out_s.at[pl.ds(obase + 15 * stripe,
                                       N_NODES - 15 * stripe)])


_seg_call = functools.partial(
    pl.kernel, mesh=_mesh,
    out_type=[jax.ShapeDtypeStruct((2 * N_NODES, H), F32)],
    scratch_types=[
        pltpu.VMEM((SEG_IG, CH), jnp.int32),
        pltpu.VMEM((SEG_IG, CH), jnp.int32),
        pltpu.VMEM((CH, H), F32),
        pltpu.VMEM((CH, H), F32),
        pltpu.VMEM((CH, H), F32),
        pltpu.VMEM((CH, H), F32),
        pltpu.VMEM((CH,), jnp.int32),
        pltpu.VMEM_SHARED((N_PAD, H), F32),
        pltpu.SemaphoreType.DMA,
        pltpu.SemaphoreType.DMA,
        pltpu.SemaphoreType.DMA,
        pltpu.SemaphoreType.DMA,
        pltpu.SemaphoreType.DMA,
        pltpu.SemaphoreType.DMA,
        pltpu.SemaphoreType.DMA,
        pltpu.SemaphoreType.DMA,
    ],
)(_seg_body)


def _cnt_body(sidx, zacc, ones_in, out_c, sv, onesv, acc, sem):
    cid = lax.axis_index("c")
    sid = lax.axis_index("s")
    base = cid * (16 * SEG_CT) + sid * SEG_CT
    stripe = N_PAD // 16
    pltpu.sync_copy(zacc.at[pl.ds(sid * stripe, stripe)],
                    acc.at[pl.ds(sid * stripe, stripe)])
    pltpu.sync_copy(ones_in, onesv)
    plsc.subcore_barrier()

    def group(g, carry):
        pltpu.sync_copy(sidx.at[pl.ds(base + g * SEG_IG, SEG_IG)], sv)

        def issue(j, carry2):
            pltpu.async_copy(onesv, acc.at[sv.at[j]], sem, add=True)
            return carry2
        lax.fori_loop(0, SEG_IG, issue, 0)

        def drain(j, carry2):
            pltpu.make_async_copy(onesv, acc.at[sv.at[j]], sem).wait()
            return carry2
        lax.fori_loop(0, SEG_IG, drain, 0)
        return carry
    lax.fori_loop(0, SEG_CT // SEG_IG, group, 0)
    plsc.subcore_barrier()
    obase = (1 - cid) * N_NODES

    @pl.when(sid < 15)
    def _():
        pltpu.sync_copy(acc.at[pl.ds(sid * stripe, stripe)],
                        out_c.at[pl.ds(obase + sid * stripe, stripe)])

    @pl.when(sid == 15)
    def _():
        pltpu.sync_copy(acc.at[pl.ds(15 * stripe, N_NODES - 15 * stripe)],
                        out_c.at[pl.ds(obase + 15 * stripe,
                                       N_NODES - 15 * stripe)])


_cnt_call = functools.partial(
    pl.kernel, mesh=_mesh,
    out_type=[jax.ShapeDtypeStruct((2 * N_NODES, H), F32)],
    scratch_types=[
        pltpu.VMEM((SEG_IG, CH), jnp.int32),
        pltpu.VMEM((CH, H), F32),
        pltpu.VMEM_SHARED((N_PAD, H), F32),
        pltpu.SemaphoreType.DMA,
    ],
)(_cnt_body)


def _cls_body(aidx, bidx, targ, out, av, bv,
              ar0, br0, ar1, br1, res, sa0, sb0, sa1, sb1):
    cid = lax.axis_index("c")
    sid = lax.axis_index("s")
    wid = cid * 16 + sid
    pltpu.sync_copy(aidx.at[pl.ds(wid * 32, 32)], av)
    pltpu.sync_copy(bidx.at[pl.ds(wid * 32, 32)], bv)
    abufs = ((ar0, br0, sa0, sb0), (ar1, br1, sa1, sb1))
    pltpu.async_copy(targ.at[av.at[0]], ar0, sa0)
    pltpu.async_copy(targ.at[bv.at[0]], br0, sb0)
    for j in range(CTL):
        ar, br, sa, sb = abufs[j % 2]
        if j + 1 < CTL:
            nar, nbr, nsa, nsb = abufs[(j + 1) % 2]
            pltpu.async_copy(targ.at[av.at[j + 1]], nar, nsa)
            pltpu.async_copy(targ.at[bv.at[j + 1]], nbr, nsb)
        pltpu.make_async_copy(targ.at[av.at[j]], ar, sa).wait()
        pltpu.make_async_copy(targ.at[bv.at[j]], br, sb).wait()

        def row(r, c2):
            acc = ar[r, pl.ds(0, 16)] * br[r, pl.ds(0, 16)]
            for k in range(1, 8):
                acc = acc + ar[r, pl.ds(k * 16, 16)] * br[r, pl.ds(k * 16, 16)]
            res[r] = acc
            return c2
        lax.fori_loop(0, E_CHUNK, row, 0)
        pltpu.sync_copy(res, out.at[pl.ds(wid * CTL * E_CHUNK + j * E_CHUNK,
                                          E_CHUNK)])


_cls_call = functools.partial(
    pl.kernel, mesh=_mesh,
    out_type=[jax.ShapeDtypeStruct((EL_PAD, 16), F32)],
    scratch_types=[
        pltpu.VMEM((32, E_CHUNK), jnp.int32),
        pltpu.VMEM((32, E_CHUNK), jnp.int32),
        pltpu.VMEM((E_CHUNK, H), F32),
        pltpu.VMEM((E_CHUNK, H), F32),
        pltpu.VMEM((E_CHUNK, H), F32),
        pltpu.VMEM((E_CHUNK, H), F32),
        pltpu.VMEM((E_CHUNK, 16), F32),
        pltpu.SemaphoreType.DMA,
        pltpu.SemaphoreType.DMA,
        pltpu.SemaphoreType.DMA,
        pltpu.SemaphoreType.DMA,
    ],
)(_cls_body)


def _cls_reduce(p):
    def body(p_ref, o_ref):
        o_ref[...] = jnp.sum(p_ref[...], axis=1)
    return pl.pallas_call(
        body,
        grid=(100,),
        in_specs=[pl.BlockSpec((1024, 16), lambda i: (i, 0))],
        out_specs=pl.BlockSpec((1024,), lambda i: (i,)),
        out_shape=jax.ShapeDtypeStruct((EL_PAD,), F32),
    )(p)


def _encode(xs, ws, bs, embs):
    def body(x_ref, w_ref, b_ref, e_ref, o_ref):
        o_ref[...] = (jnp.dot(x_ref[...], w_ref[0],
                              preferred_element_type=F32)
                      + b_ref[0] + e_ref[...])
    return pl.pallas_call(
        body,
        grid=(20,),
        in_specs=[pl.BlockSpec((1000, H), lambda i: (i, 0)),
                  pl.BlockSpec((1, H, H), lambda i: (i // 10, 0, 0)),
                  pl.BlockSpec((1, 1, H), lambda i: (i // 10, 0, 0)),
                  pl.BlockSpec((1000, H), lambda i: (i, 0))],
        out_specs=pl.BlockSpec((1000, H), lambda i: (i, 0)),
        out_shape=jax.ShapeDtypeStruct((2 * N_NODES, H), F32),
    )(xs, ws, bs, embs)


def _combine(sums, counts, h_self, wls, bls, wrs, relu):
    def body(s_ref, c_ref, h_ref, wl_ref, b_ref, wr_ref, o_ref):
        c = c_ref[:, 0]
        r = 1.0 / jnp.maximum(c, 1.0)
        agg = s_ref[...] * r[:, None]
        y = (jnp.dot(agg, wl_ref[0], preferred_element_type=F32)
             + b_ref[0]
             + jnp.dot(h_ref[...], wr_ref[0], preferred_element_type=F32))
        o_ref[...] = jnp.maximum(y, 0.0) if relu else y
    return pl.pallas_call(
        body,
        grid=(20,),
        in_specs=[pl.BlockSpec((1000, H), lambda i: (i, 0)),
                  pl.BlockSpec((1000, H), lambda i: (i, 0)),
                  pl.BlockSpec((1000, H), lambda i: (i, 0)),
                  pl.BlockSpec((1, H, H), lambda i: (i // 10, 0, 0)),
                  pl.BlockSpec((1, 1, H), lambda i: (i // 10, 0, 0)),
                  pl.BlockSpec((1, H, H), lambda i: (i // 10, 0, 0))],
        out_specs=pl.BlockSpec((1000, H), lambda i: (i, 0)),
        out_shape=jax.ShapeDtypeStruct((2 * N_NODES, H), F32),
    )(sums, counts, h_self, wls, bls, wrs)


def kernel(x_user, x_movie, user_node_id, movie_node_id, edge_index,
           edge_label_index, W_user_lin, b_user_lin, W_movie_lin, b_movie_lin,
           user_emb, movie_emb, Wl1r, bl1r, Wr1r, Wl1v, bl1v, Wr1v,
           Wl2r, bl2r, Wr2r, Wl2v, bl2v, Wr2v):
    src = edge_index[0]
    dst = edge_index[1]
    padg = jnp.zeros((E_PAD - E,), jnp.int32)
    pads = jnp.full((E_PAD - E,), N_NODES, jnp.int32)
    g1 = jnp.concatenate([src, padg])
    s1 = jnp.concatenate([dst, pads])
    g2 = jnp.concatenate([dst, padg]) + N_NODES
    s2 = jnp.concatenate([src, pads])
    g_all = jnp.concatenate([g1, g2]).reshape(2 * 16 * SEG_CT, CH)
    s_all = jnp.concatenate([s1, s2]).reshape(2 * 16 * SEG_CT, CH)
    padl = jnp.zeros((EL_PAD - EL,), jnp.int32)

    def _lidx(v):
        m = jnp.concatenate([v, padl]).reshape(N_TILES, CTL, E_CHUNK)
        return jnp.pad(m, ((0, 0), (0, 32 - CTL), (0, 0))).reshape(
            N_TILES * 32, E_CHUNK)
    la = _lidx(edge_label_index[0])
    lb = _lidx(edge_label_index[1] + N_NODES)
    zacc = jnp.zeros((N_PAD, H), F32)
    ones_rows = jnp.ones((CH, H), F32)
    padidx = jnp.full((CH,), N_NODES, jnp.int32)

    hcat0 = _encode(
        jnp.concatenate([x_user, x_movie]),
        jnp.stack([W_user_lin.T, W_movie_lin.T]),
        jnp.stack([b_user_lin.reshape(1, H), b_movie_lin.reshape(1, H)]),
        jnp.concatenate([user_emb, movie_emb]))

    (s12,) = _seg_call(g_all, s_all, hcat0, zacc, padidx)
    (c12,) = _cnt_call(s_all, zacc, ones_rows)

    hcat1 = _combine(
        s12, c12, hcat0,
        jnp.stack([Wl1v.T, Wl1r.T]),
        jnp.stack([bl1v.reshape(1, H), bl1r.reshape(1, H)]),
        jnp.stack([Wr1v.T, Wr1r.T]), True)

    (s34,) = _seg_call(g_all, s_all, hcat1, zacc, padidx)

    hcat2 = _combine(
        s34, c12, hcat1,
        jnp.stack([Wl2v.T, Wl2r.T]),
        jnp.stack([bl2v.reshape(1, H), bl2r.reshape(1, H)]),
        jnp.stack([Wr2v.T, Wr2r.T]), False)

    (partials,) = _cls_call(la, lb, hcat2)
    out = _cls_reduce(partials)
    return out[:EL]
